# initial kernel scaffold (unmeasured)
import jax
import jax.numpy as jnp
from jax import lax
from jax.experimental import pallas as pl
from jax.experimental.pallas import tpu as pltpu


def kernel(
    x,
):
    def body(*refs):
        pass

    out_shape = jax.ShapeDtypeStruct(..., jnp.float32)
    return pl.pallas_call(body, out_shape=out_shape)(...)


# baseline (device time: 463589 ns/iter reference)
import functools

import jax
import jax.numpy as jnp
from jax import lax
from jax.experimental import pallas as pl
from jax.experimental.pallas import tpu as pltpu

M = 16384
N_IN = 2048
N_OUT = 1024
HALF = M // 2


def kernel(x):

    def body(x_hbm, out_hbm, send_y, recv_all, xs, out_stage, copy_sem,
             out_sem, send_sem_y, recv_sem_y, send_sem_x, recv_sem_x):
        my_x = lax.axis_index("x")
        my_y = lax.axis_index("y")

        barrier_sem = pltpu.get_barrier_semaphore()
        pl.semaphore_signal(barrier_sem, inc=1, device_id=(my_x, 1 - my_y),
                            device_id_type=pl.DeviceIdType.MESH)
        pl.semaphore_signal(barrier_sem, inc=1, device_id=(1 - my_x, my_y),
                            device_id_type=pl.DeviceIdType.MESH)
        pl.semaphore_wait(barrier_sem, 2)

        n_sc = 8
        scm = HALF // n_sc
        for c in range(n_sc):
            cp = pltpu.make_async_copy(
                x_hbm.at[0, pl.ds(my_x * HALF + c * scm, scm),
                         pl.ds((1 - my_y) * N_OUT, N_OUT)],
                xs.at[pl.ds(0, scm), :],
                copy_sem,
            )
            cp.start()
            cp.wait()
            send_y[pl.ds(c * scm, scm), :] = xs[pl.ds(0, scm), :].astype(
                jnp.bfloat16)

        rdma_y = pltpu.make_async_remote_copy(
            src_ref=send_y,
            dst_ref=recv_all.at[pl.ds(my_x * HALF, HALF), :],
            send_sem=send_sem_y,
            recv_sem=recv_sem_y,
            device_id=(my_x, 1 - my_y),
            device_id_type=pl.DeviceIdType.MESH,
        )
        rdma_y.start()
        rdma_y.wait()

        rdma_x = pltpu.make_async_remote_copy(
            src_ref=recv_all.at[pl.ds(my_x * HALF, HALF), :],
            dst_ref=recv_all.at[pl.ds(my_x * HALF, HALF), :],
            send_sem=send_sem_x,
            recv_sem=recv_sem_x,
            device_id=(1 - my_x, my_y),
            device_id_type=pl.DeviceIdType.MESH,
        )
        rdma_x.start()
        rdma_x.wait()

        n_cc = 16
        ccm = M // n_cc
        for c in range(n_cc):
            cp = pltpu.make_async_copy(
                x_hbm.at[0, pl.ds(c * ccm, ccm), pl.ds(my_y * N_OUT, N_OUT)],
                xs.at[pl.ds(0, ccm), :],
                copy_sem,
            )
            cp.start()
            cp.wait()
            slot = c % 2
            if c >= 2:
                pltpu.make_async_copy(
                    out_stage.at[slot], out_stage.at[slot], out_sem.at[slot]
                ).wait()
            out_stage[slot] = (
                xs[pl.ds(0, ccm), :]
                + recv_all[pl.ds(c * ccm, ccm), :].astype(jnp.float32)
            ).astype(jnp.bfloat16)
            st = pltpu.make_async_copy(
                out_stage.at[slot],
                out_hbm.at[pl.ds(c * ccm, ccm), :],
                out_sem.at[slot],
            )
            st.start()
        for slot in range(2):
            pltpu.make_async_copy(
                out_stage.at[slot], out_stage.at[slot], out_sem.at[slot]
            ).wait()

        @functools.partial(pl.run_scoped,
                           second_barrier=pltpu.SemaphoreType.REGULAR)
        def _(second_barrier):
            pl.semaphore_signal(second_barrier, inc=1,
                                device_id=(my_x, 1 - my_y),
                                device_id_type=pl.DeviceIdType.MESH)
            pl.semaphore_signal(second_barrier, inc=1,
                                device_id=(1 - my_x, my_y),
                                device_id_type=pl.DeviceIdType.MESH)
            pl.semaphore_wait(second_barrier, 2)

    out_shape = jax.ShapeDtypeStruct((M, N_OUT), jnp.bfloat16)
    return pl.pallas_call(
        body,
        out_shape=out_shape,
        in_specs=[pl.BlockSpec(memory_space=pl.ANY)],
        out_specs=pl.BlockSpec(memory_space=pl.ANY),
        scratch_shapes=[
            pltpu.VMEM((HALF, N_OUT), jnp.bfloat16),
            pltpu.VMEM((M, N_OUT), jnp.bfloat16),
            pltpu.VMEM((1024, N_OUT), jnp.float32),
            pltpu.VMEM((2, 1024, N_OUT), jnp.bfloat16),
            pltpu.SemaphoreType.DMA,
            pltpu.SemaphoreType.DMA((2,)),
            pltpu.SemaphoreType.DMA,
            pltpu.SemaphoreType.DMA,
            pltpu.SemaphoreType.DMA,
            pltpu.SemaphoreType.DMA,
        ],
        compiler_params=pltpu.CompilerParams(
            collective_id=0, vmem_limit_bytes=62 * 1024 * 1024),
    )(x)


# device time: 245236 ns/iter; 1.8904x vs baseline; 1.8904x over previous
import jax
import jax.numpy as jnp
from jax import lax
from jax.experimental import pallas as pl
from jax.experimental.pallas import tpu as pltpu

M = 16384
N_IN = 2048
N_OUT = 1024
HALF = M // 2
CM = 512
K = HALF // CM


def kernel(x):

    def body(x_hbm, out_hbm, send_y, recv_all, xs, out_stage, copy_sem,
             out_sem, ysend, yrecv, xsend, xrecv):
        my_x = lax.axis_index("x")
        my_y = lax.axis_index("y")
        my_base = my_x * HALF
        other_base = (1 - my_x) * HALF

        barrier_sem = pltpu.get_barrier_semaphore()
        pl.semaphore_signal(barrier_sem, inc=1, device_id=(my_x, 1 - my_y),
                            device_id_type=pl.DeviceIdType.MESH)
        pl.semaphore_signal(barrier_sem, inc=1, device_id=(1 - my_x, my_y),
                            device_id_type=pl.DeviceIdType.MESH)
        pl.semaphore_wait(barrier_sem, 2)

        def load(row_start, col_start, slot):
            return pltpu.make_async_copy(
                x_hbm.at[0, pl.ds(row_start, CM), pl.ds(col_start, N_OUT)],
                xs.at[slot],
                copy_sem.at[slot],
            )

        def rdma_y(c):
            return pltpu.make_async_remote_copy(
                src_ref=send_y.at[pl.ds(c * CM, CM), :],
                dst_ref=recv_all.at[pl.ds(my_base + c * CM, CM), :],
                send_sem=ysend.at[c],
                recv_sem=yrecv.at[c],
                device_id=(my_x, 1 - my_y),
                device_id_type=pl.DeviceIdType.MESH,
            )

        def rdma_x(c):
            return pltpu.make_async_remote_copy(
                src_ref=recv_all.at[pl.ds(my_base + c * CM, CM), :],
                dst_ref=recv_all.at[pl.ds(my_base + c * CM, CM), :],
                send_sem=xsend.at[c],
                recv_sem=xrecv.at[c],
                device_id=(1 - my_x, my_y),
                device_id_type=pl.DeviceIdType.MESH,
            )

        def out_wait(slot):
            pltpu.make_async_copy(
                out_stage.at[slot], out_stage.at[slot], out_sem.at[slot]
            ).wait()

        loads = [load(my_base, (1 - my_y) * N_OUT, 0)]
        loads[0].start()
        for c in range(K):
            if c + 1 < K:
                nxt = load(my_base + (c + 1) * CM, (1 - my_y) * N_OUT,
                           (c + 1) % 2)
                nxt.start()
                loads.append(nxt)
            loads[c].wait()
            send_y[pl.ds(c * CM, CM), :] = xs[c % 2].astype(jnp.bfloat16)
            rdma_y(c).start()

        t = 0
        for phase in range(2):
            base = my_base if phase == 0 else other_base
            for c in range(K):
                ld = load(base + c * CM, my_y * N_OUT, t % 2)
                ld.start()
                if phase == 0:
                    rdma_y(c).wait_recv()
                    rdma_x(c).start()
                else:
                    rdma_x(c).wait_recv()
                ld.wait()
                slot = t % 2
                if t >= 2:
                    out_wait(slot)
                out_stage[slot] = (
                    xs[slot]
                    + recv_all[pl.ds(base + c * CM, CM), :].astype(jnp.float32)
                ).astype(jnp.bfloat16)
                pltpu.make_async_copy(
                    out_stage.at[slot],
                    out_hbm.at[pl.ds(base + c * CM, CM), :],
                    out_sem.at[slot],
                ).start()
                t += 1

        for slot in range(2):
            out_wait(slot)
        for c in range(K):
            rdma_y(c).wait_send()
            rdma_x(c).wait_send()

    out_shape = jax.ShapeDtypeStruct((M, N_OUT), jnp.bfloat16)
    return pl.pallas_call(
        body,
        out_shape=out_shape,
        in_specs=[pl.BlockSpec(memory_space=pl.ANY)],
        out_specs=pl.BlockSpec(memory_space=pl.ANY),
        scratch_shapes=[
            pltpu.VMEM((HALF, N_OUT), jnp.bfloat16),
            pltpu.VMEM((M, N_OUT), jnp.bfloat16),
            pltpu.VMEM((2, CM, N_OUT), jnp.float32),
            pltpu.VMEM((2, CM, N_OUT), jnp.bfloat16),
            pltpu.SemaphoreType.DMA((2,)),
            pltpu.SemaphoreType.DMA((2,)),
            pltpu.SemaphoreType.DMA((K,)),
            pltpu.SemaphoreType.DMA((K,)),
            pltpu.SemaphoreType.DMA((K,)),
            pltpu.SemaphoreType.DMA((K,)),
        ],
        compiler_params=pltpu.CompilerParams(
            collective_id=0, vmem_limit_bytes=62 * 1024 * 1024),
    )(x)
